# Initial kernel scaffold; baseline (speedup 1.0000x reference)
import jax
import jax.numpy as jnp
from jax.experimental import pallas as pl


def kernel(x, edge_index, num_nodes, W0, b0, W1, b1, W2, b2, Wc, bc):
    raise NotImplementedError("write your pallas kernel here")


# speedup vs baseline: 12.3324x; 12.3324x over previous
import functools

import jax
import jax.numpy as jnp
from jax import lax
from jax.experimental import pallas as pl
from jax.experimental.pallas import tpu as pltpu
from jax.experimental.pallas import tpu_sc as plsc

N_SC = 2
N_TILE = 16
LANES = 16
K_EDGE = 128
NPAD = 10240
D_HALF = 64
B_TC = 512


def _sc_mesh():
    return plsc.VectorSubcoreMesh(core_axis_name="c", subcore_axis_name="s")


_SC_PARAMS = pltpu.CompilerParams(use_tc_tiling_on_sc=False)


def _make_deg_kernel(e_pad):
    chunks = e_pad // (N_SC * N_TILE * K_EDGE)
    rpt = NPAD // N_TILE

    @functools.partial(
        pl.kernel,
        out_type=jax.ShapeDtypeStruct((N_SC, NPAD, LANES), jnp.float32),
        mesh=_sc_mesh(),
        compiler_params=_SC_PARAMS,
        scratch_types=[
            pltpu.VMEM_SHARED((NPAD, LANES), jnp.float32),
            pltpu.VMEM((K_EDGE,), jnp.int32),
            pltpu.VMEM((K_EDGE, LANES), jnp.float32),
        ],
    )
    def deg_kernel(rows_hbm, zeros_hbm, ones_hbm, out_hbm, dacc, idxbuf, onesbuf):
        c = lax.axis_index("c")
        s = lax.axis_index("s")
        r0 = s * rpt
        pltpu.sync_copy(zeros_hbm.at[pl.ds(r0, rpt)], dacc.at[pl.ds(r0, rpt)])
        pltpu.sync_copy(ones_hbm, onesbuf)
        plsc.subcore_barrier()
        base = (c * N_TILE + s) * (chunks * K_EDGE)

        def body(i, carry):
            off = base + i * K_EDGE
            pltpu.sync_copy(rows_hbm.at[pl.ds(off, K_EDGE)], idxbuf)
            pltpu.sync_copy(onesbuf, dacc.at[idxbuf], add=True)
            return carry

        lax.fori_loop(0, chunks, body, 0)
        plsc.subcore_barrier()
        pltpu.sync_copy(dacc.at[pl.ds(r0, rpt)], out_hbm.at[c, pl.ds(r0, rpt)])

    return deg_kernel


def _make_hop_kernel(e_pad):
    chunks = e_pad // (N_TILE * K_EDGE)
    rpt = NPAD // N_TILE

    @functools.partial(
        pl.kernel,
        out_type=jax.ShapeDtypeStruct((N_SC, NPAD, D_HALF), jnp.float32),
        mesh=_sc_mesh(),
        compiler_params=_SC_PARAMS,
        scratch_types=[
            pltpu.VMEM_SHARED((NPAD, D_HALF), jnp.float32),
            pltpu.VMEM_SHARED((NPAD, D_HALF), jnp.float32),
            pltpu.VMEM((K_EDGE,), jnp.int32),
            pltpu.VMEM((K_EDGE,), jnp.int32),
            pltpu.VMEM((K_EDGE, D_HALF), jnp.float32),
            pltpu.SemaphoreType.DMA,
        ],
    )
    def hop_kernel(u_hbm, rows_hbm, cols_hbm, out_hbm,
                   u_sp, acc_sp, colbuf, rowbuf, gbuf, sem):
        c = lax.axis_index("c")
        s = lax.axis_index("s")
        r0 = s * rpt
        pltpu.sync_copy(u_hbm.at[c, pl.ds(r0, rpt)], u_sp.at[pl.ds(r0, rpt)])
        pltpu.sync_copy(u_hbm.at[c, pl.ds(r0, rpt)], acc_sp.at[pl.ds(r0, rpt)])
        plsc.subcore_barrier()
        base = s * (chunks * K_EDGE)

        def body(i, carry):
            off = base + i * K_EDGE
            pltpu.sync_copy(cols_hbm.at[pl.ds(off, K_EDGE)], colbuf)
            pltpu.sync_copy(rows_hbm.at[pl.ds(off, K_EDGE)], rowbuf)
            pltpu.async_copy(u_sp.at[colbuf], gbuf, sem).wait()
            pltpu.sync_copy(gbuf, acc_sp.at[rowbuf], add=True)
            return carry

        lax.fori_loop(0, chunks, body, 0)
        plsc.subcore_barrier()
        pltpu.sync_copy(acc_sp.at[pl.ds(r0, rpt)], out_hbm.at[c, pl.ds(r0, rpt)])

    return hop_kernel


def _tc_pre():
    def body(x_ref, d_ref, u_ref, dis_ref):
        deg = d_ref[0, :, :1] + d_ref[1, :, :1] + 1.0
        dis = lax.rsqrt(deg)
        us = x_ref[...] * dis
        u_ref[...] = jnp.stack([us[:, :D_HALF], us[:, D_HALF:]], axis=0)
        dis_ref[...] = jnp.broadcast_to(dis, (B_TC, LANES))

    return pl.pallas_call(
        body,
        grid=(NPAD // B_TC,),
        in_specs=[
            pl.BlockSpec((B_TC, 128), lambda i: (i, 0)),
            pl.BlockSpec((N_SC, B_TC, LANES), lambda i: (0, i, 0)),
        ],
        out_specs=[
            pl.BlockSpec((N_SC, B_TC, D_HALF), lambda i: (0, i, 0)),
            pl.BlockSpec((B_TC, LANES), lambda i: (i, 0)),
        ],
        out_shape=[
            jax.ShapeDtypeStruct((N_SC, NPAD, D_HALF), jnp.float32),
            jax.ShapeDtypeStruct((NPAD, LANES), jnp.float32),
        ],
    )


def _tc_mid():
    def body(t_ref, dis_ref, u2_ref):
        dis = dis_ref[:, :1]
        u2_ref[...] = t_ref[...] * (dis * dis)[None]

    return pl.pallas_call(
        body,
        grid=(NPAD // B_TC,),
        in_specs=[
            pl.BlockSpec((N_SC, B_TC, D_HALF), lambda i: (0, i, 0)),
            pl.BlockSpec((B_TC, LANES), lambda i: (i, 0)),
        ],
        out_specs=pl.BlockSpec((N_SC, B_TC, D_HALF), lambda i: (0, i, 0)),
        out_shape=jax.ShapeDtypeStruct((N_SC, NPAD, D_HALF), jnp.float32),
    )


def _tc_post():
    def body(x_ref, t1_ref, t2_ref, dis_ref, w0_ref, w1_ref, w2_ref,
             wc_ref, bias_ref, out_ref):
        f32 = jnp.float32
        dis = dis_ref[:, :1]
        y1 = jnp.concatenate([t1_ref[0], t1_ref[1]], axis=1) * dis
        y2 = jnp.concatenate([t2_ref[0], t2_ref[1]], axis=1) * dis
        h0 = jax.nn.relu(
            jnp.dot(x_ref[...], w0_ref[...], preferred_element_type=f32)
            + bias_ref[0:1])
        h1 = jax.nn.relu(
            jnp.dot(y1, w1_ref[...], preferred_element_type=f32)
            + bias_ref[1:2])
        h2 = jax.nn.relu(
            jnp.dot(y2, w2_ref[...], preferred_element_type=f32)
            + bias_ref[2:3])
        out_ref[...] = (
            jnp.dot(h0, wc_ref[0:128], preferred_element_type=f32)
            + jnp.dot(h1, wc_ref[128:256], preferred_element_type=f32)
            + jnp.dot(h2, wc_ref[256:384], preferred_element_type=f32)
            + bias_ref[3:4])

    return pl.pallas_call(
        body,
        grid=(NPAD // B_TC,),
        in_specs=[
            pl.BlockSpec((B_TC, 128), lambda i: (i, 0)),
            pl.BlockSpec((N_SC, B_TC, D_HALF), lambda i: (0, i, 0)),
            pl.BlockSpec((N_SC, B_TC, D_HALF), lambda i: (0, i, 0)),
            pl.BlockSpec((B_TC, LANES), lambda i: (i, 0)),
            pl.BlockSpec((128, 128), lambda i: (0, 0)),
            pl.BlockSpec((128, 128), lambda i: (0, 0)),
            pl.BlockSpec((128, 128), lambda i: (0, 0)),
            pl.BlockSpec((384, 128), lambda i: (0, 0)),
            pl.BlockSpec((4, 128), lambda i: (0, 0)),
        ],
        out_specs=pl.BlockSpec((B_TC, 128), lambda i: (i, 0)),
        out_shape=jax.ShapeDtypeStruct((NPAD, 128), jnp.float32),
    )


def kernel(x, edge_index, num_nodes, W0, b0, W1, b1, W2, b2, Wc, bc):
    n, d_in = x.shape
    e = edge_index.shape[1]
    grp = N_SC * N_TILE * K_EDGE
    e_pad = ((e + grp - 1) // grp) * grp

    nn1 = jnp.asarray(num_nodes, jnp.int32) - 1
    row = jnp.minimum(edge_index[0].astype(jnp.int32), nn1)
    col = jnp.minimum(edge_index[1].astype(jnp.int32), nn1)
    pad = jnp.full((e_pad - e,), n, jnp.int32)
    rows = jnp.concatenate([row, pad])
    cols = jnp.concatenate([col, pad])

    x_pad = jnp.zeros((NPAD, d_in), jnp.float32).at[:n].set(x)
    zeros16 = jnp.zeros((NPAD, LANES), jnp.float32)
    ones16 = jnp.ones((K_EDGE, LANES), jnp.float32)

    dparts = _make_deg_kernel(e_pad)(rows, zeros16, ones16)
    u1, dis = _tc_pre()(x_pad, dparts)
    hop = _make_hop_kernel(e_pad)
    t1 = hop(u1, rows, cols)
    u2 = _tc_mid()(t1, dis)
    t2 = hop(u2, rows, cols)
    bias = jnp.stack([b0, b1, b2, bc])
    out = _tc_post()(x_pad, t1, t2, dis, W0.T, W1.T, W2.T, Wc.T, bias)
    return out[:n]


# speedup vs baseline: 20.9127x; 1.6958x over previous
import functools

import jax
import jax.numpy as jnp
from jax import lax
from jax.experimental import pallas as pl
from jax.experimental.pallas import tpu as pltpu
from jax.experimental.pallas import tpu_sc as plsc

N_SC = 2
N_TILE = 16
LANES = 16
K_EDGE = 128
GB = 8
NPAD = 10240
D_HALF = 64
B_TC = 512

_SC_PARAMS = pltpu.CompilerParams(use_tc_tiling_on_sc=False)


def _sc_mesh():
    return plsc.VectorSubcoreMesh(core_axis_name="c", subcore_axis_name="s")


def _make_deg_kernel(e_pad):
    chunks = e_pad // (N_SC * N_TILE * K_EDGE)
    nbatch = chunks // GB
    assert chunks % GB == 0 and nbatch % 2 == 0
    rpt = NPAD // N_TILE

    @functools.partial(
        pl.kernel,
        out_type=jax.ShapeDtypeStruct((N_SC, NPAD, LANES), jnp.float32),
        mesh=_sc_mesh(),
        compiler_params=_SC_PARAMS,
        scratch_types=[
            pltpu.VMEM_SHARED((NPAD, LANES), jnp.float32),
            pltpu.VMEM((2, GB, K_EDGE), jnp.int32),
            pltpu.VMEM((K_EDGE, LANES), jnp.float32),
            pltpu.SemaphoreType.DMA,
            pltpu.SemaphoreType.DMA,
            pltpu.SemaphoreType.DMA,
        ],
    )
    def deg_kernel(rows_hbm, zeros_hbm, ones_hbm, out_hbm, dacc, idxb, onesb,
                   isem0, isem1, ssem):
        c = lax.axis_index("c")
        s = lax.axis_index("s")
        r0 = s * rpt
        pltpu.sync_copy(zeros_hbm.at[pl.ds(r0, rpt)], dacc.at[pl.ds(r0, rpt)])
        pltpu.sync_copy(ones_hbm, onesb)
        plsc.subcore_barrier()
        cbase = (c * N_TILE + s) * chunks
        isems = (isem0, isem1)

        def idx_load(bi, slot):
            row0 = cbase + bi * GB
            pltpu.async_copy(rows_hbm.at[pl.ds(row0, GB)], idxb.at[slot],
                             isems[slot])

        def wait_idx(slot):
            pltpu.make_async_copy(rows_hbm.at[pl.ds(0, GB)], idxb.at[slot],
                                  isems[slot]).wait()

        def drain_scatters(n):
            for _ in range(n):
                pltpu.make_async_copy(onesb, dacc.at[idxb.at[0, 0]],
                                      ssem).wait()

        idx_load(0, 0)

        def do_batch(bi, slot, first):
            wait_idx(slot)
            if not first:
                drain_scatters(GB)

            @pl.when(bi + 1 < nbatch)
            def _():
                idx_load(bi + 1, 1 - slot)
            for j in range(GB):
                pltpu.async_copy(onesb, dacc.at[idxb.at[slot, j]], ssem,
                                 add=True)

        do_batch(0, 0, True)
        do_batch(1, 1, False)

        def body(p, carry):
            do_batch(2 * p, 0, False)
            do_batch(2 * p + 1, 1, False)
            return carry

        lax.fori_loop(1, nbatch // 2, body, 0)
        drain_scatters(GB)
        plsc.subcore_barrier()
        pltpu.sync_copy(dacc.at[pl.ds(r0, rpt)], out_hbm.at[c, pl.ds(r0, rpt)])

    return deg_kernel


def _make_hop_kernel(e_pad):
    chunks = e_pad // (N_TILE * K_EDGE)
    nbatch = chunks // GB
    assert chunks % GB == 0 and nbatch % 2 == 0 and GB % 2 == 0
    rpt = NPAD // N_TILE

    @functools.partial(
        pl.kernel,
        out_type=jax.ShapeDtypeStruct((N_SC, NPAD, D_HALF), jnp.float32),
        mesh=_sc_mesh(),
        compiler_params=_SC_PARAMS,
        scratch_types=[
            pltpu.VMEM_SHARED((NPAD, D_HALF), jnp.float32),
            pltpu.VMEM_SHARED((NPAD, D_HALF), jnp.float32),
            pltpu.VMEM((2, GB, K_EDGE), jnp.int32),
            pltpu.VMEM((2, GB, K_EDGE), jnp.int32),
            pltpu.VMEM((2, K_EDGE, D_HALF), jnp.float32),
            pltpu.SemaphoreType.DMA,
            pltpu.SemaphoreType.DMA,
            pltpu.SemaphoreType.DMA,
            pltpu.SemaphoreType.DMA,
            pltpu.SemaphoreType.DMA,
        ],
    )
    def hop_kernel(u_hbm, rows_hbm, cols_hbm, out_hbm,
                   u_sp, acc_sp, colb, rowb, gb,
                   isem0, isem1, gsem, ssem0, ssem1):
        c = lax.axis_index("c")
        s = lax.axis_index("s")
        r0 = s * rpt
        pltpu.sync_copy(u_hbm.at[c, pl.ds(r0, rpt)], u_sp.at[pl.ds(r0, rpt)])
        pltpu.sync_copy(u_hbm.at[c, pl.ds(r0, rpt)], acc_sp.at[pl.ds(r0, rpt)])
        plsc.subcore_barrier()
        cbase = s * chunks
        isems = (isem0, isem1)
        ssems = (ssem0, ssem1)

        def idx_load(bi, slot):
            row0 = cbase + bi * GB
            pltpu.async_copy(cols_hbm.at[pl.ds(row0, GB)], colb.at[slot],
                             isems[slot])
            pltpu.async_copy(rows_hbm.at[pl.ds(row0, GB)], rowb.at[slot],
                             isems[slot])

        def wait_idx(slot):
            pltpu.make_async_copy(cols_hbm.at[pl.ds(0, GB)], colb.at[slot],
                                  isems[slot]).wait()
            pltpu.make_async_copy(rows_hbm.at[pl.ds(0, GB)], rowb.at[slot],
                                  isems[slot]).wait()

        def drain_scatter(b):
            pltpu.make_async_copy(gb.at[b], acc_sp.at[rowb.at[0, 0]],
                                  ssems[b]).wait()

        idx_load(0, 0)

        def do_batch(bi, slot, first):
            wait_idx(slot)
            if not first:
                drain_scatter(0)
                drain_scatter(1)

            @pl.when(bi + 1 < nbatch)
            def _():
                idx_load(bi + 1, 1 - slot)
            for j in range(GB):
                b = j % 2
                if j >= 2:
                    drain_scatter(b)
                pltpu.async_copy(u_sp.at[colb.at[slot, j]], gb.at[b],
                                 gsem).wait()
                pltpu.async_copy(gb.at[b], acc_sp.at[rowb.at[slot, j]],
                                 ssems[b], add=True)

        do_batch(0, 0, True)
        do_batch(1, 1, False)

        def body(p, carry):
            do_batch(2 * p, 0, False)
            do_batch(2 * p + 1, 1, False)
            return carry

        lax.fori_loop(1, nbatch // 2, body, 0)
        drain_scatter(0)
        drain_scatter(1)
        plsc.subcore_barrier()
        pltpu.sync_copy(acc_sp.at[pl.ds(r0, rpt)], out_hbm.at[c, pl.ds(r0, rpt)])

    return hop_kernel


def _tc_pre():
    def body(x_ref, d_ref, u_ref, dis_ref):
        deg = d_ref[0, :, :1] + d_ref[1, :, :1] + 1.0
        dis = lax.rsqrt(deg)
        us = x_ref[...] * dis
        u_ref[...] = jnp.stack([us[:, :D_HALF], us[:, D_HALF:]], axis=0)
        dis_ref[...] = jnp.broadcast_to(dis, (B_TC, LANES))

    return pl.pallas_call(
        body,
        grid=(NPAD // B_TC,),
        in_specs=[
            pl.BlockSpec((B_TC, 128), lambda i: (i, 0)),
            pl.BlockSpec((N_SC, B_TC, LANES), lambda i: (0, i, 0)),
        ],
        out_specs=[
            pl.BlockSpec((N_SC, B_TC, D_HALF), lambda i: (0, i, 0)),
            pl.BlockSpec((B_TC, LANES), lambda i: (i, 0)),
        ],
        out_shape=[
            jax.ShapeDtypeStruct((N_SC, NPAD, D_HALF), jnp.float32),
            jax.ShapeDtypeStruct((NPAD, LANES), jnp.float32),
        ],
    )


def _tc_mid():
    def body(t_ref, dis_ref, u2_ref):
        dis = dis_ref[:, :1]
        u2_ref[...] = t_ref[...] * (dis * dis)[None]

    return pl.pallas_call(
        body,
        grid=(NPAD // B_TC,),
        in_specs=[
            pl.BlockSpec((N_SC, B_TC, D_HALF), lambda i: (0, i, 0)),
            pl.BlockSpec((B_TC, LANES), lambda i: (i, 0)),
        ],
        out_specs=pl.BlockSpec((N_SC, B_TC, D_HALF), lambda i: (0, i, 0)),
        out_shape=jax.ShapeDtypeStruct((N_SC, NPAD, D_HALF), jnp.float32),
    )


def _tc_post():
    def body(x_ref, t1_ref, t2_ref, dis_ref, w0_ref, w1_ref, w2_ref,
             wc_ref, bias_ref, out_ref):
        f32 = jnp.float32
        dis = dis_ref[:, :1]
        y1 = jnp.concatenate([t1_ref[0], t1_ref[1]], axis=1) * dis
        y2 = jnp.concatenate([t2_ref[0], t2_ref[1]], axis=1) * dis
        h0 = jax.nn.relu(
            jnp.dot(x_ref[...], w0_ref[...], preferred_element_type=f32)
            + bias_ref[0:1])
        h1 = jax.nn.relu(
            jnp.dot(y1, w1_ref[...], preferred_element_type=f32)
            + bias_ref[1:2])
        h2 = jax.nn.relu(
            jnp.dot(y2, w2_ref[...], preferred_element_type=f32)
            + bias_ref[2:3])
        out_ref[...] = (
            jnp.dot(h0, wc_ref[0:128], preferred_element_type=f32)
            + jnp.dot(h1, wc_ref[128:256], preferred_element_type=f32)
            + jnp.dot(h2, wc_ref[256:384], preferred_element_type=f32)
            + bias_ref[3:4])

    return pl.pallas_call(
        body,
        grid=(NPAD // B_TC,),
        in_specs=[
            pl.BlockSpec((B_TC, 128), lambda i: (i, 0)),
            pl.BlockSpec((N_SC, B_TC, D_HALF), lambda i: (0, i, 0)),
            pl.BlockSpec((N_SC, B_TC, D_HALF), lambda i: (0, i, 0)),
            pl.BlockSpec((B_TC, LANES), lambda i: (i, 0)),
            pl.BlockSpec((128, 128), lambda i: (0, 0)),
            pl.BlockSpec((128, 128), lambda i: (0, 0)),
            pl.BlockSpec((128, 128), lambda i: (0, 0)),
            pl.BlockSpec((384, 128), lambda i: (0, 0)),
            pl.BlockSpec((4, 128), lambda i: (0, 0)),
        ],
        out_specs=pl.BlockSpec((B_TC, 128), lambda i: (i, 0)),
        out_shape=jax.ShapeDtypeStruct((NPAD, 128), jnp.float32),
    )


def kernel(x, edge_index, num_nodes, W0, b0, W1, b1, W2, b2, Wc, bc):
    n, d_in = x.shape
    e = edge_index.shape[1]
    grp = N_SC * N_TILE * K_EDGE * GB * 2
    e_pad = ((e + grp - 1) // grp) * grp

    nn1 = jnp.asarray(num_nodes, jnp.int32) - 1
    row = jnp.minimum(edge_index[0].astype(jnp.int32), nn1)
    col = jnp.minimum(edge_index[1].astype(jnp.int32), nn1)
    pad = jnp.full((e_pad - e,), n, jnp.int32)
    rows = jnp.concatenate([row, pad]).reshape(e_pad // K_EDGE, K_EDGE)
    cols = jnp.concatenate([col, pad]).reshape(e_pad // K_EDGE, K_EDGE)

    x_pad = jnp.zeros((NPAD, d_in), jnp.float32).at[:n].set(x)
    zeros16 = jnp.zeros((NPAD, LANES), jnp.float32)
    ones16 = jnp.ones((K_EDGE, LANES), jnp.float32)

    dparts = _make_deg_kernel(e_pad)(rows, zeros16, ones16)
    u1, dis = _tc_pre()(x_pad, dparts)
    hop = _make_hop_kernel(e_pad)
    t1 = hop(u1, rows, cols)
    u2 = _tc_mid()(t1, dis)
    t2 = hop(u2, rows, cols)
    bias = jnp.stack([b0, b1, b2, bc])
    out = _tc_post()(x_pad, t1, t2, dis, W0.T, W1.T, W2.T, Wc.T, bias)
    return out[:n]


# speedup vs baseline: 21.0916x; 1.0086x over previous
import functools

import jax
import jax.numpy as jnp
from jax import lax
from jax.experimental import pallas as pl
from jax.experimental.pallas import tpu as pltpu
from jax.experimental.pallas import tpu_sc as plsc

N_SC = 2
N_TILE = 16
LANES = 16
K_EDGE = 128
GB = 8
NPAD = 10240
D_HALF = 64
B_TC = 512
NB_G = 4

_SC_PARAMS = pltpu.CompilerParams(use_tc_tiling_on_sc=False)


def _sc_mesh():
    return plsc.VectorSubcoreMesh(core_axis_name="c", subcore_axis_name="s")


def _make_deg_kernel(e_pad):
    chunks = e_pad // (N_SC * N_TILE * K_EDGE)
    nbatch = chunks // GB
    assert chunks % GB == 0 and nbatch % 2 == 0
    rpt = NPAD // N_TILE

    @functools.partial(
        pl.kernel,
        out_type=jax.ShapeDtypeStruct((N_SC, NPAD, LANES), jnp.float32),
        mesh=_sc_mesh(),
        compiler_params=_SC_PARAMS,
        scratch_types=[
            pltpu.VMEM_SHARED((NPAD, LANES), jnp.float32),
            pltpu.VMEM((2, GB, K_EDGE), jnp.int32),
            pltpu.VMEM((K_EDGE, LANES), jnp.float32),
            pltpu.SemaphoreType.DMA,
            pltpu.SemaphoreType.DMA,
            pltpu.SemaphoreType.DMA,
        ],
    )
    def deg_kernel(rows_hbm, zeros_hbm, ones_hbm, out_hbm, dacc, idxb, onesb,
                   isem0, isem1, ssem):
        c = lax.axis_index("c")
        s = lax.axis_index("s")
        r0 = s * rpt
        pltpu.sync_copy(zeros_hbm.at[pl.ds(r0, rpt)], dacc.at[pl.ds(r0, rpt)])
        pltpu.sync_copy(ones_hbm, onesb)
        plsc.subcore_barrier()
        cbase = (c * N_TILE + s) * chunks
        isems = (isem0, isem1)

        def idx_load(bi, slot):
            row0 = cbase + bi * GB
            pltpu.async_copy(rows_hbm.at[pl.ds(row0, GB)], idxb.at[slot],
                             isems[slot])

        def wait_idx(slot):
            pltpu.make_async_copy(rows_hbm.at[pl.ds(0, GB)], idxb.at[slot],
                                  isems[slot]).wait()

        def drain_scatters(n):
            for _ in range(n):
                pltpu.make_async_copy(onesb, dacc.at[idxb.at[0, 0]],
                                      ssem).wait()

        idx_load(0, 0)

        def do_batch(bi, slot, first):
            wait_idx(slot)
            if not first:
                drain_scatters(GB)

            @pl.when(bi + 1 < nbatch)
            def _():
                idx_load(bi + 1, 1 - slot)
            for j in range(GB):
                pltpu.async_copy(onesb, dacc.at[idxb.at[slot, j]], ssem,
                                 add=True)

        do_batch(0, 0, True)
        do_batch(1, 1, False)

        def body(p, carry):
            do_batch(2 * p, 0, False)
            do_batch(2 * p + 1, 1, False)
            return carry

        lax.fori_loop(1, nbatch // 2, body, 0)
        drain_scatters(GB)
        plsc.subcore_barrier()
        pltpu.sync_copy(dacc.at[pl.ds(r0, rpt)], out_hbm.at[c, pl.ds(r0, rpt)])

    return deg_kernel


def _make_hop_kernel(e_pad):
    chunks = e_pad // (N_TILE * K_EDGE)
    nbatch = chunks // GB
    assert chunks % GB == 0 and nbatch % 2 == 0 and GB % NB_G == 0
    rpt = NPAD // N_TILE

    @functools.partial(
        pl.kernel,
        out_type=jax.ShapeDtypeStruct((N_SC, NPAD, D_HALF), jnp.float32),
        mesh=_sc_mesh(),
        compiler_params=_SC_PARAMS,
        scratch_types=[
            pltpu.VMEM_SHARED((NPAD, D_HALF), jnp.float32),
            pltpu.VMEM_SHARED((NPAD, D_HALF), jnp.float32),
            pltpu.VMEM((2, GB, K_EDGE), jnp.int32),
            pltpu.VMEM((2, GB, K_EDGE), jnp.int32),
            pltpu.VMEM((NB_G, K_EDGE, D_HALF), jnp.float32),
            pltpu.SemaphoreType.DMA,
            pltpu.SemaphoreType.DMA,
            pltpu.SemaphoreType.DMA,
            pltpu.SemaphoreType.DMA,
            pltpu.SemaphoreType.DMA,
            pltpu.SemaphoreType.DMA,
            pltpu.SemaphoreType.DMA,
            pltpu.SemaphoreType.DMA,
            pltpu.SemaphoreType.DMA,
            pltpu.SemaphoreType.DMA,
        ],
    )
    def hop_kernel(u_hbm, rows_hbm, cols_hbm, out_hbm,
                   u_sp, acc_sp, colb, rowb, gb,
                   isem0, isem1, g0, g1, g2, g3, s0, s1, s2, s3):
        c = lax.axis_index("c")
        s = lax.axis_index("s")
        r0 = s * rpt
        pltpu.sync_copy(u_hbm.at[c, pl.ds(r0, rpt)], u_sp.at[pl.ds(r0, rpt)])
        pltpu.sync_copy(u_hbm.at[c, pl.ds(r0, rpt)], acc_sp.at[pl.ds(r0, rpt)])
        plsc.subcore_barrier()
        cbase = s * chunks
        isems = (isem0, isem1)
        gsems = (g0, g1, g2, g3)
        ssems = (s0, s1, s2, s3)

        def idx_load(bi, slot):
            row0 = cbase + bi * GB
            pltpu.async_copy(cols_hbm.at[pl.ds(row0, GB)], colb.at[slot],
                             isems[slot])
            pltpu.async_copy(rows_hbm.at[pl.ds(row0, GB)], rowb.at[slot],
                             isems[slot])

        def wait_idx(slot):
            pltpu.make_async_copy(cols_hbm.at[pl.ds(0, GB)], colb.at[slot],
                                  isems[slot]).wait()
            pltpu.make_async_copy(rows_hbm.at[pl.ds(0, GB)], rowb.at[slot],
                                  isems[slot]).wait()

        def gather(slot, j):
            b = j % NB_G
            pltpu.async_copy(u_sp.at[colb.at[slot, j]], gb.at[b], gsems[b])

        def wait_gather(j):
            b = j % NB_G
            pltpu.make_async_copy(u_sp.at[colb.at[0, 0]], gb.at[b],
                                  gsems[b]).wait()

        def scatter(slot, j):
            b = j % NB_G
            pltpu.async_copy(gb.at[b], acc_sp.at[rowb.at[slot, j]], ssems[b],
                             add=True)

        def drain_scatter(b):
            pltpu.make_async_copy(gb.at[b], acc_sp.at[rowb.at[0, 0]],
                                  ssems[b]).wait()

        idx_load(0, 0)

        def do_batch(bi, slot, first):
            wait_idx(slot)
            if not first:
                for b in range(NB_G):
                    drain_scatter(b)

            @pl.when(bi + 1 < nbatch)
            def _():
                idx_load(bi + 1, 1 - slot)
            gather(slot, 0)
            gather(slot, 1)
            for j in range(GB):
                wait_gather(j)
                scatter(slot, j)
                if j + 2 < GB:
                    if j >= 2:
                        drain_scatter((j + 2) % NB_G)
                    gather(slot, j + 2)

        do_batch(0, 0, True)
        do_batch(1, 1, False)

        def body(p, carry):
            do_batch(2 * p, 0, False)
            do_batch(2 * p + 1, 1, False)
            return carry

        lax.fori_loop(1, nbatch // 2, body, 0)
        for b in range(NB_G):
            drain_scatter(b)
        plsc.subcore_barrier()
        pltpu.sync_copy(acc_sp.at[pl.ds(r0, rpt)], out_hbm.at[c, pl.ds(r0, rpt)])

    return hop_kernel


def _tc_pre():
    def body(x_ref, d_ref, u_ref, dis_ref):
        deg = d_ref[0, :, :1] + d_ref[1, :, :1] + 1.0
        dis = lax.rsqrt(deg)
        us = x_ref[...] * dis
        u_ref[...] = jnp.stack([us[:, :D_HALF], us[:, D_HALF:]], axis=0)
        dis_ref[...] = jnp.broadcast_to(dis, (B_TC, LANES))

    return pl.pallas_call(
        body,
        grid=(NPAD // B_TC,),
        in_specs=[
            pl.BlockSpec((B_TC, 128), lambda i: (i, 0)),
            pl.BlockSpec((N_SC, B_TC, LANES), lambda i: (0, i, 0)),
        ],
        out_specs=[
            pl.BlockSpec((N_SC, B_TC, D_HALF), lambda i: (0, i, 0)),
            pl.BlockSpec((B_TC, LANES), lambda i: (i, 0)),
        ],
        out_shape=[
            jax.ShapeDtypeStruct((N_SC, NPAD, D_HALF), jnp.float32),
            jax.ShapeDtypeStruct((NPAD, LANES), jnp.float32),
        ],
    )


def _tc_mid():
    def body(t_ref, dis_ref, u2_ref):
        dis = dis_ref[:, :1]
        u2_ref[...] = t_ref[...] * (dis * dis)[None]

    return pl.pallas_call(
        body,
        grid=(NPAD // B_TC,),
        in_specs=[
            pl.BlockSpec((N_SC, B_TC, D_HALF), lambda i: (0, i, 0)),
            pl.BlockSpec((B_TC, LANES), lambda i: (i, 0)),
        ],
        out_specs=pl.BlockSpec((N_SC, B_TC, D_HALF), lambda i: (0, i, 0)),
        out_shape=jax.ShapeDtypeStruct((N_SC, NPAD, D_HALF), jnp.float32),
    )


def _tc_post():
    def body(x_ref, t1_ref, t2_ref, dis_ref, w0_ref, w1_ref, w2_ref,
             wc_ref, bias_ref, out_ref):
        f32 = jnp.float32
        dis = dis_ref[:, :1]
        y1 = jnp.concatenate([t1_ref[0], t1_ref[1]], axis=1) * dis
        y2 = jnp.concatenate([t2_ref[0], t2_ref[1]], axis=1) * dis
        h0 = jax.nn.relu(
            jnp.dot(x_ref[...], w0_ref[...], preferred_element_type=f32)
            + bias_ref[0:1])
        h1 = jax.nn.relu(
            jnp.dot(y1, w1_ref[...], preferred_element_type=f32)
            + bias_ref[1:2])
        h2 = jax.nn.relu(
            jnp.dot(y2, w2_ref[...], preferred_element_type=f32)
            + bias_ref[2:3])
        out_ref[...] = (
            jnp.dot(h0, wc_ref[0:128], preferred_element_type=f32)
            + jnp.dot(h1, wc_ref[128:256], preferred_element_type=f32)
            + jnp.dot(h2, wc_ref[256:384], preferred_element_type=f32)
            + bias_ref[3:4])

    return pl.pallas_call(
        body,
        grid=(NPAD // B_TC,),
        in_specs=[
            pl.BlockSpec((B_TC, 128), lambda i: (i, 0)),
            pl.BlockSpec((N_SC, B_TC, D_HALF), lambda i: (0, i, 0)),
            pl.BlockSpec((N_SC, B_TC, D_HALF), lambda i: (0, i, 0)),
            pl.BlockSpec((B_TC, LANES), lambda i: (i, 0)),
            pl.BlockSpec((128, 128), lambda i: (0, 0)),
            pl.BlockSpec((128, 128), lambda i: (0, 0)),
            pl.BlockSpec((128, 128), lambda i: (0, 0)),
            pl.BlockSpec((384, 128), lambda i: (0, 0)),
            pl.BlockSpec((4, 128), lambda i: (0, 0)),
        ],
        out_specs=pl.BlockSpec((B_TC, 128), lambda i: (i, 0)),
        out_shape=jax.ShapeDtypeStruct((NPAD, 128), jnp.float32),
    )


def kernel(x, edge_index, num_nodes, W0, b0, W1, b1, W2, b2, Wc, bc):
    n, d_in = x.shape
    e = edge_index.shape[1]
    grp = N_SC * N_TILE * K_EDGE * GB * 2
    e_pad = ((e + grp - 1) // grp) * grp

    nn1 = jnp.asarray(num_nodes, jnp.int32) - 1
    row = jnp.minimum(edge_index[0].astype(jnp.int32), nn1)
    col = jnp.minimum(edge_index[1].astype(jnp.int32), nn1)
    pad = jnp.full((e_pad - e,), n, jnp.int32)
    rows = jnp.concatenate([row, pad]).reshape(e_pad // K_EDGE, K_EDGE)
    cols = jnp.concatenate([col, pad]).reshape(e_pad // K_EDGE, K_EDGE)

    x_pad = jnp.zeros((NPAD, d_in), jnp.float32).at[:n].set(x)
    zeros16 = jnp.zeros((NPAD, LANES), jnp.float32)
    ones16 = jnp.ones((K_EDGE, LANES), jnp.float32)

    dparts = _make_deg_kernel(e_pad)(rows, zeros16, ones16)
    u1, dis = _tc_pre()(x_pad, dparts)
    hop = _make_hop_kernel(e_pad)
    t1 = hop(u1, rows, cols)
    u2 = _tc_mid()(t1, dis)
    t2 = hop(u2, rows, cols)
    bias = jnp.stack([b0, b1, b2, bc])
    out = _tc_post()(x_pad, t1, t2, dis, W0.T, W1.T, W2.T, Wc.T, bias)
    return out[:n]
